# baseline (speedup 1.0000x reference)
import functools

import jax
import jax.numpy as jnp
from jax import lax
from jax.experimental import pallas as pl
from jax.experimental.pallas import tpu as pltpu
from jax.experimental.pallas import tpu_sc as plsc

N = 10000
E = 320000
D = 128
IRR = 25
BW = 25
TW = 32

_NC = 2
_NS = 16
_NW = _NC * _NS
_EW = E // _NW
_IW = 125
_CH = 1000
_CR = _CH // _IW
_G = _EW // _CH
_NCHUNK = 624
_NODE_WORKERS = 16
_NTAIL = N - _NCHUNK * _NODE_WORKERS


def _tc_body(x_ref, wn_ref, we_ref, cobn_ref, cobe_ref,
             node_ref, p_ref, q_ref):
    cobn = cobn_ref[...]
    cobe = cobe_ref[...]
    we = we_ref[...]
    m = jnp.dot(wn_ref[...], cobn, preferred_element_type=jnp.float32)
    a = jnp.dot(we[:D, :], cobe, preferred_element_type=jnp.float32)
    b = jnp.dot(we[D:, :], cobe, preferred_element_type=jnp.float32)
    x = x_ref[...]
    node_ref[...] = jnp.dot(x, m, preferred_element_type=jnp.float32)
    p_ref[...] = jnp.dot(x, a, preferred_element_type=jnp.float32)
    q_ref[...] = jnp.dot(x, b, preferred_element_type=jnp.float32)


_ROWS_PER_BLK = 1000

_tc_matmul = pl.pallas_call(
    _tc_body,
    grid=(N // _ROWS_PER_BLK,),
    in_specs=[
        pl.BlockSpec((_ROWS_PER_BLK, D), lambda i: (i, 0)),
        pl.BlockSpec((D, IRR), lambda i: (0, 0)),
        pl.BlockSpec((2 * D, IRR), lambda i: (0, 0)),
        pl.BlockSpec((IRR, BW), lambda i: (0, 0)),
        pl.BlockSpec((IRR, TW), lambda i: (0, 0)),
    ],
    out_specs=[
        pl.BlockSpec((_ROWS_PER_BLK, BW), lambda i: (i, 0)),
        pl.BlockSpec((_ROWS_PER_BLK, TW), lambda i: (i, 0)),
        pl.BlockSpec((_ROWS_PER_BLK, TW), lambda i: (i, 0)),
    ],
    out_shape=[
        jax.ShapeDtypeStruct((N, BW), jnp.float32),
        jax.ShapeDtypeStruct((N, TW), jnp.float32),
        jax.ShapeDtypeStruct((N, TW), jnp.float32),
    ],
)


_sc_mesh = plsc.VectorSubcoreMesh(core_axis_name="c", subcore_axis_name="s")


@functools.partial(
    pl.kernel,
    mesh=_sc_mesh,
    out_type=jax.ShapeDtypeStruct((E, TW), jnp.float32),
    compiler_params=pltpu.CompilerParams(use_tc_tiling_on_sc=False),
    scratch_types=[
        pltpu.VMEM((_CR, _IW), jnp.int32),
        pltpu.VMEM((_CR, _IW), jnp.int32),
        pltpu.VMEM((_CH, TW), jnp.float32),
        pltpu.VMEM((_CH, TW), jnp.float32),
        pltpu.VMEM((_CH, TW), jnp.float32),
        pltpu.SemaphoreType.DMA,
    ],
)
def _sc_edge(p_hbm, q_hbm, src_hbm, dst_hbm, out_hbm,
             src_v, dst_v, rows_p, rows_q, out_v, sem):
    wid = lax.axis_index("s") * _NC + lax.axis_index("c")
    row0 = wid * (_EW // _IW)

    def chunk_body(g, _):
        r0 = row0 + g * _CR
        pltpu.sync_copy(src_hbm.at[pl.ds(r0, _CR)], src_v)
        pltpu.sync_copy(dst_hbm.at[pl.ds(r0, _CR)], dst_v)
        copies = []
        for j in range(_CR):
            copies.append(pltpu.async_copy(
                p_hbm.at[src_v.at[j]], rows_p.at[pl.ds(j * _IW, _IW)], sem))
            copies.append(pltpu.async_copy(
                q_hbm.at[dst_v.at[j]], rows_q.at[pl.ds(j * _IW, _IW)], sem))
        for c in copies:
            c.wait()

        def add_body(r8, _):
            r = r8 * 8
            for u in range(8):
                lo = rows_p[r + u, pl.ds(0, 16)] + rows_q[r + u, pl.ds(0, 16)]
                hi = rows_p[r + u, pl.ds(16, 16)] + rows_q[r + u, pl.ds(16, 16)]
                out_v[r + u, pl.ds(0, 16)] = lo
                out_v[r + u, pl.ds(16, 16)] = hi
            return 0

        lax.fori_loop(0, _CH // 8, add_body, 0)
        e0 = wid * _EW + g * _CH
        pltpu.sync_copy(out_v, out_hbm.at[pl.ds(e0, _CH)])
        return 0

    lax.fori_loop(0, _G, chunk_body, 0)


def kernel(node_feats, W_node, W_edge, cob_node, cob_edge, edge_index):
    cobn = cob_node.reshape(IRR, BW)
    cobe = cob_edge.reshape(IRR, BW)
    cobe_pad = jnp.zeros((IRR, TW), jnp.float32).at[:, :BW].set(cobe)
    node_out, p32, q32 = _tc_matmul(node_feats, W_node, W_edge, cobn, cobe_pad)
    src2d = edge_index[0].reshape(E // _IW, _IW)
    dst2d = edge_index[1].reshape(E // _IW, _IW)
    edge32 = _sc_edge(p32, q32, src2d, dst2d)
    return jnp.concatenate([node_out, edge32[:, :BW]], axis=0)


# speedup vs baseline: 1.9245x; 1.9245x over previous
import functools

import jax
import jax.numpy as jnp
from jax import lax
from jax.experimental import pallas as pl
from jax.experimental.pallas import tpu as pltpu
from jax.experimental.pallas import tpu_sc as plsc

N = 10000
E = 320000
D = 128
IRR = 25
BW = 25
TW = 32

_NC = 2
_NS = 16
_NW = _NC * _NS
_IW = 128
_CR = 4
_CH = _CR * _IW
_NCHK = E // _CH
_SLOTS = 20
_EPAD = _SLOTS * _NW * _CH
_EB = _CH // 16
_NODE_CH = 312
_NTAIL = N - _NODE_CH * _NW


def _tc_body(x_ref, wn_ref, we_ref, cobn_ref, cobe_ref,
             nodet_ref, p_ref, q_ref):
    cobn = cobn_ref[...]
    cobe = cobe_ref[...]
    we = we_ref[...]
    m = jnp.dot(wn_ref[...], cobn, preferred_element_type=jnp.float32)
    a = jnp.dot(we[:D, :], cobe, preferred_element_type=jnp.float32)
    b = jnp.dot(we[D:, :], cobe, preferred_element_type=jnp.float32)
    x = x_ref[...]
    nodet_ref[...] = jnp.dot(x, jnp.pad(m, ((0, 0), (0, TW - BW))),
                             preferred_element_type=jnp.float32)
    p_ref[...] = jnp.dot(x, a, preferred_element_type=jnp.float32)
    q_ref[...] = jnp.dot(x, b, preferred_element_type=jnp.float32)


_tc_matmul = pl.pallas_call(
    _tc_body,
    out_shape=[
        jax.ShapeDtypeStruct((N, TW), jnp.float32),
        jax.ShapeDtypeStruct((N, TW), jnp.float32),
        jax.ShapeDtypeStruct((N, TW), jnp.float32),
    ],
)


_sc_mesh = plsc.VectorSubcoreMesh(core_axis_name="c", subcore_axis_name="s")


@functools.partial(
    pl.kernel,
    mesh=_sc_mesh,
    out_type=jax.ShapeDtypeStruct((N + E, 128), jnp.float32),
    compiler_params=pltpu.CompilerParams(use_tc_tiling_on_sc=False),
    scratch_types=[
        pltpu.VMEM((_CR, _IW), jnp.int32),
        pltpu.VMEM((_CR, _IW), jnp.int32),
        pltpu.VMEM((_CR, _IW), jnp.int32),
        pltpu.VMEM((_CR, _IW), jnp.int32),
        pltpu.VMEM((_CH, TW), jnp.float32),
        pltpu.VMEM((_CH, TW), jnp.float32),
        pltpu.VMEM((_CH, TW), jnp.float32),
        pltpu.VMEM((_CH, TW), jnp.float32),
        pltpu.VMEM((_CH, TW), jnp.float32),
        pltpu.VMEM((_CH, TW), jnp.float32),
        pltpu.VMEM((_NODE_CH, TW), jnp.float32),
        pltpu.SemaphoreType.DMA,
        pltpu.SemaphoreType.DMA,
        pltpu.SemaphoreType.DMA,
        pltpu.SemaphoreType.DMA,
        pltpu.SemaphoreType.DMA,
        pltpu.SemaphoreType.DMA,
    ],
)
def _sc_edge(node_hbm, p_hbm, q_hbm, src_hbm, dst_hbm, out_hbm,
             src0, src1, dst0, dst1, rp0, rp1, rq0, rq1, ot0, ot1,
             node_v, semi0, semi1, semg0, semg1, semo0, semo1):
    wid = lax.axis_index("s") * _NC + lax.axis_index("c")
    srcb, dstb = (src0, src1), (dst0, dst1)
    rpb, rqb, otb = (rp0, rp1), (rq0, rq1), (ot0, ot1)
    semi, semg, semo = (semi0, semi1), (semg0, semg1), (semo0, semo1)

    def cid_of(k):
        raw = k * _NW + wid
        return jnp.where(raw < _NCHK, raw, raw - _NW)

    def fire_idx(k):
        b = k % 2
        r0 = cid_of(k) * _CR
        return [pltpu.async_copy(src_hbm.at[pl.ds(r0, _CR)], srcb[b], semi[b]),
                pltpu.async_copy(dst_hbm.at[pl.ds(r0, _CR)], dstb[b], semi[b])]

    def fire_gathers(k):
        b = k % 2
        hs = []
        for j in range(_CR):
            hs.append(pltpu.async_copy(
                p_hbm.at[srcb[b].at[j]],
                rpb[b].at[pl.ds(j * _IW, _IW)], semg[b]))
            hs.append(pltpu.async_copy(
                q_hbm.at[dstb[b].at[j]],
                rqb[b].at[pl.ds(j * _IW, _IW)], semg[b]))
        return hs

    def fire_out(k):
        b = k % 2
        r0 = N + cid_of(k) * _CH
        return [pltpu.async_copy(
            otb[b], out_hbm.at[pl.ds(r0, _CH), pl.ds(0, TW)], semo[b])]

    def add_rows(k):
        b = k % 2
        rp, rq, ot = rpb[b], rqb[b], otb[b]

        def eblk(i, _):
            r = i * 8
            for u in range(8):
                lo = rp[r + u, pl.ds(0, 16)] + rq[r + u, pl.ds(0, 16)]
                hi = rp[r + u, pl.ds(16, 16)] + rq[r + u, pl.ds(16, 16)]
                ot[r + u, pl.ds(0, 16)] = lo
                ot[r + u, pl.ds(16, 16)] = hi
            return 0

        lax.fori_loop(0, _CH // 8, eblk, 0)

    idx_h = {0: fire_idx(0), 1: fire_idx(1)}
    for h in idx_h[0]:
        h.wait()
    gath_h = {0: fire_gathers(0)}
    out_h = {}

    nb = wid * _NODE_CH
    pltpu.sync_copy(node_hbm.at[pl.ds(nb, _NODE_CH)], node_v)
    pltpu.sync_copy(node_v, out_hbm.at[pl.ds(nb, _NODE_CH), pl.ds(0, TW)])

    @pl.when(wid == 0)
    def _():
        tb = _NW * _NODE_CH
        tail = node_v.at[pl.ds(0, _NTAIL)]
        pltpu.sync_copy(node_hbm.at[pl.ds(tb, _NTAIL)], tail)
        pltpu.sync_copy(tail, out_hbm.at[pl.ds(tb, _NTAIL), pl.ds(0, TW)])

    for k in range(_SLOTS):
        if k + 1 < _SLOTS:
            for h in idx_h.pop(k + 1):
                h.wait()
            gath_h[k + 1] = fire_gathers(k + 1)
        for h in gath_h.pop(k):
            h.wait()
        if k + 2 < _SLOTS:
            idx_h[k + 2] = fire_idx(k + 2)
        if k - 2 in out_h:
            for h in out_h.pop(k - 2):
                h.wait()
        add_rows(k)
        out_h[k] = fire_out(k)

    for k in sorted(out_h):
        for h in out_h.pop(k):
            h.wait()


def kernel(node_feats, W_node, W_edge, cob_node, cob_edge, edge_index):
    cobn = cob_node.reshape(IRR, BW)
    cobe = cob_edge.reshape(IRR, BW)
    cobe_pad = jnp.zeros((IRR, TW), jnp.float32).at[:, :BW].set(cobe)
    node32, p32, q32 = _tc_matmul(node_feats, W_node, W_edge, cobn,
                                  cobe_pad)
    src2d = jnp.pad(edge_index[0], (0, _EPAD - E)).reshape(_EPAD // _IW, _IW)
    dst2d = jnp.pad(edge_index[1], (0, _EPAD - E)).reshape(_EPAD // _IW, _IW)
    out128 = _sc_edge(node32, p32, q32, src2d, dst2d)
    return out128[:, :BW]
